# Initial kernel scaffold; baseline (speedup 1.0000x reference)
import jax
import jax.numpy as jnp
from jax.experimental import pallas as pl


def kernel(x, ln1_g, ln1_b, Wqkv, bqkv, Wo, bo, ln2_g, ln2_b, Wr, W1, b1, W2, b2):
    raise NotImplementedError("write your pallas kernel here")


# speedup vs baseline: 1.0096x; 1.0096x over previous
import functools

import jax
import jax.numpy as jnp
from jax import lax
from jax.experimental import pallas as pl
from jax.experimental.pallas import tpu as pltpu

_B, _S, _D = 2, 2048, 1024
_H, _Dh = 16, 64
_E, _K, _F = 8, 2, 4096
_T = _B * _S
_BH = _B * _H

_RT = 256
_BQ = 256
_TILE = 256
_P = _K * _T + _E * _TILE
_NT = _P // _TILE
_FB = 2048
_NF = _F // _FB


def _ln(x, g, b):
    mu = jnp.mean(x, axis=-1, keepdims=True)
    xc = x - mu
    var = jnp.mean(xc * xc, axis=-1, keepdims=True)
    return xc * lax.rsqrt(var + 1e-5) * g + b


def _ln_qkv_body(x_ref, g_ref, b_ref, w_ref, bias_ref, o_ref):
    h = _ln(x_ref[...], g_ref[...], b_ref[...])
    o_ref[...] = (jnp.dot(h, w_ref[...], preferred_element_type=jnp.float32)
                  + bias_ref[...])


def _ln_qkv(x2d, g, b, w, bias):
    return pl.pallas_call(
        _ln_qkv_body,
        grid=(_T // _RT,),
        in_specs=[
            pl.BlockSpec((_RT, _D), lambda i: (i, 0)),
            pl.BlockSpec((1, _D), lambda i: (0, 0)),
            pl.BlockSpec((1, _D), lambda i: (0, 0)),
            pl.BlockSpec((_D, 3 * _D), lambda i: (0, 0)),
            pl.BlockSpec((1, 3 * _D), lambda i: (0, 0)),
        ],
        out_specs=pl.BlockSpec((_RT, 3 * _D), lambda i: (i, 0)),
        out_shape=jax.ShapeDtypeStruct((_T, 3 * _D), jnp.float32),
    )(x2d, g.reshape(1, _D), b.reshape(1, _D), w, bias.reshape(1, 3 * _D))


def _attn_body(q_ref, k_ref, v_ref, o_ref):
    qi = pl.program_id(1)
    q = q_ref[0]
    k = k_ref[0]
    s = lax.dot_general(q, k, (((1,), (1,)), ((), ())),
                        preferred_element_type=jnp.float32) * 0.125
    row = qi * _BQ + lax.broadcasted_iota(jnp.int32, (_BQ, _S), 0)
    col = lax.broadcasted_iota(jnp.int32, (_BQ, _S), 1)
    s = jnp.where(col <= row, s, jnp.float32(-1e9))
    m = jnp.max(s, axis=-1, keepdims=True)
    p = jnp.exp(s - m)
    p = p / jnp.sum(p, axis=-1, keepdims=True)
    o_ref[0] = jnp.dot(p, v_ref[0], preferred_element_type=jnp.float32)


def _attention(q, k, v):
    return pl.pallas_call(
        _attn_body,
        grid=(_BH, _S // _BQ),
        in_specs=[
            pl.BlockSpec((1, _BQ, _Dh), lambda bh, i: (bh, i, 0)),
            pl.BlockSpec((1, _S, _Dh), lambda bh, i: (bh, 0, 0)),
            pl.BlockSpec((1, _S, _Dh), lambda bh, i: (bh, 0, 0)),
        ],
        out_specs=pl.BlockSpec((1, _BQ, _Dh), lambda bh, i: (bh, i, 0)),
        out_shape=jax.ShapeDtypeStruct((_BH, _S, _Dh), jnp.float32),
    )(q, k, v)


def _proj_route_body(o_ref, xres_ref, wo_ref, bo_ref, g_ref, b_ref, wr_ref,
                     x2_ref, h2_ref, rt_ref):
    x2 = (jnp.dot(o_ref[...], wo_ref[...], preferred_element_type=jnp.float32)
          + bo_ref[...] + xres_ref[...])
    x2_ref[...] = x2
    h2 = _ln(x2, g_ref[...], b_ref[...])
    h2_ref[...] = h2
    logits = jnp.dot(h2, wr_ref[...], preferred_element_type=jnp.float32)
    col = lax.broadcasted_iota(jnp.int32, (_RT, 128), 1)
    lg = jnp.where(col < _E, logits, jnp.float32(-1e30))
    m = jnp.max(lg, axis=-1, keepdims=True)
    p = jnp.exp(lg - m)
    p = jnp.where(col < _E, p, 0.0)
    p = p / jnp.sum(p, axis=-1, keepdims=True)
    big = jnp.int32(1 << 30)
    m1 = jnp.max(p, axis=-1, keepdims=True)
    i1 = jnp.min(jnp.where(p == m1, col, big), axis=-1, keepdims=True)
    p2 = jnp.where(col == i1, jnp.float32(-1.0), p)
    m2 = jnp.max(p2, axis=-1, keepdims=True)
    i2 = jnp.min(jnp.where(p2 == m2, col, big), axis=-1, keepdims=True)
    den = m1 + m2
    g1 = m1 / den
    g2 = m2 / den
    rt = jnp.where(col == 0, i1.astype(jnp.float32),
         jnp.where(col == 1, i2.astype(jnp.float32),
         jnp.where(col == 2, g1,
         jnp.where(col == 3, g2, jnp.float32(0.0)))))
    rt_ref[...] = rt


def _proj_route(o2d, x2d, wo, bo, g, b, wr_pad):
    return pl.pallas_call(
        _proj_route_body,
        grid=(_T // _RT,),
        in_specs=[
            pl.BlockSpec((_RT, _D), lambda i: (i, 0)),
            pl.BlockSpec((_RT, _D), lambda i: (i, 0)),
            pl.BlockSpec((_D, _D), lambda i: (0, 0)),
            pl.BlockSpec((1, _D), lambda i: (0, 0)),
            pl.BlockSpec((1, _D), lambda i: (0, 0)),
            pl.BlockSpec((1, _D), lambda i: (0, 0)),
            pl.BlockSpec((_D, 128), lambda i: (0, 0)),
        ],
        out_specs=[
            pl.BlockSpec((_RT, _D), lambda i: (i, 0)),
            pl.BlockSpec((_RT, _D), lambda i: (i, 0)),
            pl.BlockSpec((_RT, 128), lambda i: (i, 0)),
        ],
        out_shape=[
            jax.ShapeDtypeStruct((_T, _D), jnp.float32),
            jax.ShapeDtypeStruct((_T, _D), jnp.float32),
            jax.ShapeDtypeStruct((_T, 128), jnp.float32),
        ],
    )(o2d, x2d, wo, bo.reshape(1, _D), g.reshape(1, _D), b.reshape(1, _D),
      wr_pad)


def _moe_body(te_ref, x_ref, w1_ref, b1_ref, w2_ref, b2_ref, gate_ref, y_ref):
    f = pl.program_id(1)
    h = (jnp.dot(x_ref[...], w1_ref[0], preferred_element_type=jnp.float32)
         + b1_ref[0])
    h = jax.nn.gelu(h)
    y = jnp.dot(h, w2_ref[0], preferred_element_type=jnp.float32)

    @pl.when(f == 0)
    def _():
        y_ref[...] = (y + b2_ref[0]) * gate_ref[0]

    @pl.when(f != 0)
    def _():
        y_ref[...] += y * gate_ref[0]


def _moe_ffn(xs, w1, b1, w2, b2, gates, tile_expert):
    grid_spec = pltpu.PrefetchScalarGridSpec(
        num_scalar_prefetch=1,
        grid=(_NT, _NF),
        in_specs=[
            pl.BlockSpec((_TILE, _D), lambda i, f, te: (i, 0)),
            pl.BlockSpec((1, _D, _FB), lambda i, f, te: (te[i], 0, f)),
            pl.BlockSpec((1, 1, _FB), lambda i, f, te: (te[i], 0, f)),
            pl.BlockSpec((1, _FB, _D), lambda i, f, te: (te[i], f, 0)),
            pl.BlockSpec((1, 1, _D), lambda i, f, te: (te[i], 0, 0)),
            pl.BlockSpec((1, _TILE, 1), lambda i, f, te: (i, 0, 0)),
        ],
        out_specs=pl.BlockSpec((_TILE, _D), lambda i, f, te: (i, 0)),
    )
    return pl.pallas_call(
        _moe_body,
        grid_spec=grid_spec,
        out_shape=jax.ShapeDtypeStruct((_P, _D), jnp.float32),
    )(tile_expert, xs, w1, b1.reshape(_E, 1, _F), w2, b2.reshape(_E, 1, _D),
      gates.reshape(_NT, _TILE, 1))


def kernel(x, ln1_g, ln1_b, Wqkv, bqkv, Wo, bo, ln2_g, ln2_b, Wr, W1, b1,
           W2, b2):
    x2d = x.reshape(_T, _D)

    qkv = _ln_qkv(x2d, ln1_g, ln1_b, Wqkv, bqkv)
    q, k, v = qkv[:, :_D], qkv[:, _D:2 * _D], qkv[:, 2 * _D:]

    def to_heads(t):
        return (t.reshape(_B, _S, _H, _Dh).transpose(0, 2, 1, 3)
                .reshape(_BH, _S, _Dh))

    o = _attention(to_heads(q), to_heads(k), to_heads(v))
    o2d = (o.reshape(_B, _H, _S, _Dh).transpose(0, 2, 1, 3)
           .reshape(_T, _D))

    wr_pad = jnp.zeros((_D, 128), jnp.float32).at[:, :_E].set(Wr)
    x2, h2, route = _proj_route(o2d, x2d, Wo, bo, ln2_g, ln2_b, wr_pad)

    i1 = route[:, 0].astype(jnp.int32)
    i2 = route[:, 1].astype(jnp.int32)
    g1 = route[:, 2]
    g2 = route[:, 3]

    e_pairs = jnp.stack([i1, i2], axis=1).reshape(-1)
    g_pairs = jnp.stack([g1, g2], axis=1).reshape(-1)
    order = jnp.argsort(e_pairs, stable=True)
    e_sorted = e_pairs[order]
    counts = jnp.sum(e_pairs[:, None] == jnp.arange(_E)[None, :], axis=0)
    rc = ((counts + _TILE - 1) // _TILE) * _TILE
    ps = jnp.concatenate([jnp.zeros((1,), jnp.int32),
                          jnp.cumsum(rc)[:-1].astype(jnp.int32)])
    su = (jnp.cumsum(counts) - counts).astype(jnp.int32)
    rank = jnp.arange(_K * _T, dtype=jnp.int32) - su[e_sorted]
    dest = ps[e_sorted] + rank
    tok_sorted = (order // _K).astype(jnp.int32)
    pos = jnp.zeros((_K * _T,), jnp.int32).at[order].set(dest)
    p1 = pos[0::2]
    p2 = pos[1::2]
    tok_pad = jnp.zeros((_P,), jnp.int32).at[dest].set(tok_sorted)
    gate_pad = jnp.zeros((_P,), jnp.float32).at[dest].set(g_pairs[order])
    tile_starts = jnp.arange(_NT, dtype=jnp.int32) * _TILE
    tile_expert = (jnp.sum(ps[None, :] <= tile_starts[:, None], axis=1)
                   .astype(jnp.int32) - 1).clip(0, _E - 1)

    xs = jnp.take(h2, tok_pad, axis=0)
    ys = _moe_ffn(xs, W1, b1, W2, b2, gate_pad, tile_expert)
    out = x2 + jnp.take(ys, p1, axis=0) + jnp.take(ys, p2, axis=0)
    return out.reshape(_B, _S, _D)


# speedup vs baseline: 1.0767x; 1.0666x over previous
import functools

import jax
import jax.numpy as jnp
from jax import lax
from jax.experimental import pallas as pl
from jax.experimental.pallas import tpu as pltpu

_B, _S, _D = 2, 2048, 1024
_H, _Dh = 16, 64
_E, _K, _F = 8, 2, 4096
_T = _B * _S
_BH = _B * _H

_RT = 256
_BQ = 256
_TILE = 256
_P = _K * _T + _E * _TILE
_NT = _P // _TILE
_FB = 2048
_NF = _F // _FB


def _ln(x, g, b):
    mu = jnp.mean(x, axis=-1, keepdims=True)
    xc = x - mu
    var = jnp.mean(xc * xc, axis=-1, keepdims=True)
    return xc * lax.rsqrt(var + 1e-5) * g + b


def _ln_qkv_body(x_ref, g_ref, b_ref, w_ref, bias_ref, o_ref):
    h = _ln(x_ref[...], g_ref[...], b_ref[...])
    o_ref[...] = (jnp.dot(h, w_ref[...], preferred_element_type=jnp.float32)
                  + bias_ref[...])


def _ln_qkv(x2d, g, b, w, bias):
    return pl.pallas_call(
        _ln_qkv_body,
        grid=(_T // _RT,),
        in_specs=[
            pl.BlockSpec((_RT, _D), lambda i: (i, 0)),
            pl.BlockSpec((1, _D), lambda i: (0, 0)),
            pl.BlockSpec((1, _D), lambda i: (0, 0)),
            pl.BlockSpec((_D, 3 * _D), lambda i: (0, 0)),
            pl.BlockSpec((1, 3 * _D), lambda i: (0, 0)),
        ],
        out_specs=pl.BlockSpec((_RT, 3 * _D), lambda i: (i, 0)),
        out_shape=jax.ShapeDtypeStruct((_T, 3 * _D), jnp.float32),
    )(x2d, g.reshape(1, _D), b.reshape(1, _D), w, bias.reshape(1, 3 * _D))


def _attn_body(q_ref, k_ref, v_ref, o_ref):
    qi = pl.program_id(1)
    q = q_ref[0]
    k = k_ref[0]
    s = lax.dot_general(q, k, (((1,), (1,)), ((), ())),
                        preferred_element_type=jnp.float32) * 0.125
    row = qi * _BQ + lax.broadcasted_iota(jnp.int32, (_BQ, _S), 0)
    col = lax.broadcasted_iota(jnp.int32, (_BQ, _S), 1)
    s = jnp.where(col <= row, s, jnp.float32(-1e9))
    m = jnp.max(s, axis=-1, keepdims=True)
    p = jnp.exp(s - m)
    p = p / jnp.sum(p, axis=-1, keepdims=True)
    o_ref[0] = jnp.dot(p, v_ref[0], preferred_element_type=jnp.float32)


def _attention(q, k, v):
    return pl.pallas_call(
        _attn_body,
        grid=(_BH, _S // _BQ),
        in_specs=[
            pl.BlockSpec((1, _BQ, _Dh), lambda bh, i: (bh, i, 0)),
            pl.BlockSpec((1, _S, _Dh), lambda bh, i: (bh, 0, 0)),
            pl.BlockSpec((1, _S, _Dh), lambda bh, i: (bh, 0, 0)),
        ],
        out_specs=pl.BlockSpec((1, _BQ, _Dh), lambda bh, i: (bh, i, 0)),
        out_shape=jax.ShapeDtypeStruct((_BH, _S, _Dh), jnp.float32),
    )(q, k, v)


def _proj_route_body(o_ref, xres_ref, wo_ref, bo_ref, g_ref, b_ref, wr_ref,
                     x2_ref, h2_ref, rt_ref):
    x2 = (jnp.dot(o_ref[...], wo_ref[...], preferred_element_type=jnp.float32)
          + bo_ref[...] + xres_ref[...])
    x2_ref[...] = x2
    h2 = _ln(x2, g_ref[...], b_ref[...])
    h2_ref[...] = h2
    logits = jnp.dot(h2, wr_ref[...], preferred_element_type=jnp.float32)
    col = lax.broadcasted_iota(jnp.int32, (_RT, 128), 1)
    lg = jnp.where(col < _E, logits, jnp.float32(-1e30))
    m = jnp.max(lg, axis=-1, keepdims=True)
    p = jnp.exp(lg - m)
    p = jnp.where(col < _E, p, 0.0)
    p = p / jnp.sum(p, axis=-1, keepdims=True)
    big = jnp.int32(1 << 30)
    m1 = jnp.max(p, axis=-1, keepdims=True)
    i1 = jnp.min(jnp.where(p == m1, col, big), axis=-1, keepdims=True)
    p2 = jnp.where(col == i1, jnp.float32(-1.0), p)
    m2 = jnp.max(p2, axis=-1, keepdims=True)
    i2 = jnp.min(jnp.where(p2 == m2, col, big), axis=-1, keepdims=True)
    den = m1 + m2
    g1 = m1 / den
    g2 = m2 / den
    rt = jnp.where(col == 0, i1.astype(jnp.float32),
         jnp.where(col == 1, i2.astype(jnp.float32),
         jnp.where(col == 2, g1,
         jnp.where(col == 3, g2, jnp.float32(0.0)))))
    rt_ref[...] = rt


def _proj_route(o2d, x2d, wo, bo, g, b, wr_pad):
    return pl.pallas_call(
        _proj_route_body,
        grid=(_T // _RT,),
        in_specs=[
            pl.BlockSpec((_RT, _D), lambda i: (i, 0)),
            pl.BlockSpec((_RT, _D), lambda i: (i, 0)),
            pl.BlockSpec((_D, _D), lambda i: (0, 0)),
            pl.BlockSpec((1, _D), lambda i: (0, 0)),
            pl.BlockSpec((1, _D), lambda i: (0, 0)),
            pl.BlockSpec((1, _D), lambda i: (0, 0)),
            pl.BlockSpec((_D, 128), lambda i: (0, 0)),
        ],
        out_specs=[
            pl.BlockSpec((_RT, _D), lambda i: (i, 0)),
            pl.BlockSpec((_RT, _D), lambda i: (i, 0)),
            pl.BlockSpec((_RT, 128), lambda i: (i, 0)),
        ],
        out_shape=[
            jax.ShapeDtypeStruct((_T, _D), jnp.float32),
            jax.ShapeDtypeStruct((_T, _D), jnp.float32),
            jax.ShapeDtypeStruct((_T, 128), jnp.float32),
        ],
    )(o2d, x2d, wo, bo.reshape(1, _D), g.reshape(1, _D), b.reshape(1, _D),
      wr_pad)


def _moe_body(te_ref, x_ref, w1_ref, b1_ref, w2_ref, b2_ref, gate_ref, y_ref):
    h = (jnp.dot(x_ref[...], w1_ref[0], preferred_element_type=jnp.float32)
         + b1_ref[0])
    h = jax.nn.gelu(h).astype(jnp.bfloat16)
    y = jnp.dot(h, w2_ref[0], preferred_element_type=jnp.float32)
    y_ref[...] = (y + b2_ref[0]) * gate_ref[0]


def _moe_ffn(xs, w1, b1, w2, b2, gates, tile_expert):
    grid_spec = pltpu.PrefetchScalarGridSpec(
        num_scalar_prefetch=1,
        grid=(_NT,),
        in_specs=[
            pl.BlockSpec((_TILE, _D), lambda i, te: (i, 0)),
            pl.BlockSpec((1, _D, _F), lambda i, te: (te[i], 0, 0)),
            pl.BlockSpec((1, 1, _F), lambda i, te: (te[i], 0, 0)),
            pl.BlockSpec((1, _F, _D), lambda i, te: (te[i], 0, 0)),
            pl.BlockSpec((1, 1, _D), lambda i, te: (te[i], 0, 0)),
            pl.BlockSpec((1, _TILE, 1), lambda i, te: (i, 0, 0)),
        ],
        out_specs=pl.BlockSpec((_TILE, _D), lambda i, te: (i, 0)),
    )
    return pl.pallas_call(
        _moe_body,
        grid_spec=grid_spec,
        out_shape=jax.ShapeDtypeStruct((_P, _D), jnp.float32),
        compiler_params=pltpu.CompilerParams(
            vmem_limit_bytes=100 * 1024 * 1024),
    )(tile_expert, xs.astype(jnp.bfloat16), w1.astype(jnp.bfloat16),
      b1.reshape(_E, 1, _F), w2.astype(jnp.bfloat16),
      b2.reshape(_E, 1, _D), gates.reshape(_NT, _TILE, 1))


def kernel(x, ln1_g, ln1_b, Wqkv, bqkv, Wo, bo, ln2_g, ln2_b, Wr, W1, b1,
           W2, b2):
    x2d = x.reshape(_T, _D)

    qkv = _ln_qkv(x2d, ln1_g, ln1_b, Wqkv, bqkv)
    q, k, v = qkv[:, :_D], qkv[:, _D:2 * _D], qkv[:, 2 * _D:]

    def to_heads(t):
        return (t.reshape(_B, _S, _H, _Dh).transpose(0, 2, 1, 3)
                .reshape(_BH, _S, _Dh))

    o = _attention(to_heads(q), to_heads(k), to_heads(v))
    o2d = (o.reshape(_B, _H, _S, _Dh).transpose(0, 2, 1, 3)
           .reshape(_T, _D))

    wr_pad = jnp.zeros((_D, 128), jnp.float32).at[:, :_E].set(Wr)
    x2, h2, route = _proj_route(o2d, x2d, Wo, bo, ln2_g, ln2_b, wr_pad)

    i1 = route[:, 0].astype(jnp.int32)
    i2 = route[:, 1].astype(jnp.int32)
    g1 = route[:, 2]
    g2 = route[:, 3]

    e_pairs = jnp.stack([i1, i2], axis=1).reshape(-1)
    g_pairs = jnp.stack([g1, g2], axis=1).reshape(-1)
    order = jnp.argsort(e_pairs, stable=True)
    e_sorted = e_pairs[order]
    counts = jnp.sum(e_pairs[:, None] == jnp.arange(_E)[None, :], axis=0)
    rc = ((counts + _TILE - 1) // _TILE) * _TILE
    ps = jnp.concatenate([jnp.zeros((1,), jnp.int32),
                          jnp.cumsum(rc)[:-1].astype(jnp.int32)])
    su = (jnp.cumsum(counts) - counts).astype(jnp.int32)
    rank = jnp.arange(_K * _T, dtype=jnp.int32) - su[e_sorted]
    dest = ps[e_sorted] + rank
    tok_sorted = (order // _K).astype(jnp.int32)
    pos = jnp.zeros((_K * _T,), jnp.int32).at[order].set(dest)
    p1 = pos[0::2]
    p2 = pos[1::2]
    tok_pad = jnp.zeros((_P,), jnp.int32).at[dest].set(tok_sorted)
    gate_pad = jnp.zeros((_P,), jnp.float32).at[dest].set(g_pairs[order])
    tile_starts = jnp.arange(_NT, dtype=jnp.int32) * _TILE
    tile_expert = (jnp.sum(ps[None, :] <= tile_starts[:, None], axis=1)
                   .astype(jnp.int32) - 1).clip(0, _E - 1)

    xs = jnp.take(h2, tok_pad, axis=0)
    ys = _moe_ffn(xs, W1, b1, W2, b2, gate_pad, tile_expert)
    out = x2 + jnp.take(ys, p1, axis=0) + jnp.take(ys, p2, axis=0)
    return out.reshape(_B, _S, _D)


# speedup vs baseline: 1.2355x; 1.1474x over previous
import functools

import jax
import jax.numpy as jnp
from jax import lax
from jax.experimental import pallas as pl
from jax.experimental.pallas import tpu as pltpu

_B, _S, _D = 2, 2048, 1024
_H, _Dh = 16, 64
_E, _K, _F = 8, 2, 4096
_T = _B * _S
_BH = _B * _H

_RT = 256
_BQ = 256
_TILE = 256
_P = _K * _T + _E * _TILE
_NT = _P // _TILE
_FB = 2048
_NF = _F // _FB


def _ln(x, g, b):
    mu = jnp.mean(x, axis=-1, keepdims=True)
    xc = x - mu
    var = jnp.mean(xc * xc, axis=-1, keepdims=True)
    return xc * lax.rsqrt(var + 1e-5) * g + b


def _ln_qkv_body(x_ref, g_ref, b_ref, w_ref, bias_ref, o_ref):
    h = _ln(x_ref[...], g_ref[...], b_ref[...])
    o_ref[...] = (jnp.dot(h, w_ref[...], preferred_element_type=jnp.float32)
                  + bias_ref[...])


def _ln_qkv(x2d, g, b, w, bias):
    return pl.pallas_call(
        _ln_qkv_body,
        grid=(_T // _RT,),
        in_specs=[
            pl.BlockSpec((_RT, _D), lambda i: (i, 0)),
            pl.BlockSpec((1, _D), lambda i: (0, 0)),
            pl.BlockSpec((1, _D), lambda i: (0, 0)),
            pl.BlockSpec((_D, 3 * _D), lambda i: (0, 0)),
            pl.BlockSpec((1, 3 * _D), lambda i: (0, 0)),
        ],
        out_specs=pl.BlockSpec((_RT, 3 * _D), lambda i: (i, 0)),
        out_shape=jax.ShapeDtypeStruct((_T, 3 * _D), jnp.float32),
    )(x2d, g.reshape(1, _D), b.reshape(1, _D), w, bias.reshape(1, 3 * _D))


def _attn_body(q_ref, k_ref, v_ref, o_ref):
    qi = pl.program_id(2)
    row = qi * _BQ + lax.broadcasted_iota(jnp.int32, (_BQ, _S), 0)
    col = lax.broadcasted_iota(jnp.int32, (_BQ, _S), 1)
    causal = col <= row
    for u in range(2):
        sl = pl.ds(u * _Dh, _Dh)
        q = q_ref[:, sl]
        k = k_ref[:, sl]
        s = lax.dot_general(q, k, (((1,), (1,)), ((), ())),
                            preferred_element_type=jnp.float32) * 0.125
        s = jnp.where(causal, s, jnp.float32(-1e9))
        m = jnp.max(s, axis=-1, keepdims=True)
        p = jnp.exp(s - m)
        p = p / jnp.sum(p, axis=-1, keepdims=True)
        o_ref[:, sl] = jnp.dot(p, v_ref[:, sl],
                               preferred_element_type=jnp.float32)


def _attention(qkv):
    nq = _S // _BQ
    nhp = _H // 2
    return pl.pallas_call(
        _attn_body,
        grid=(_B, nhp, nq),
        in_specs=[
            pl.BlockSpec((_BQ, 128), lambda b, hp, qi: (b * nq + qi, hp)),
            pl.BlockSpec((_S, 128), lambda b, hp, qi: (b, nhp + hp)),
            pl.BlockSpec((_S, 128), lambda b, hp, qi: (b, 2 * nhp + hp)),
        ],
        out_specs=pl.BlockSpec((_BQ, 128), lambda b, hp, qi: (b * nq + qi, hp)),
        out_shape=jax.ShapeDtypeStruct((_T, _D), jnp.float32),
    )(qkv, qkv, qkv)


def _proj_route_body(o_ref, xres_ref, wo_ref, bo_ref, g_ref, b_ref, wr_ref,
                     x2_ref, h2_ref, rt_ref):
    x2 = (jnp.dot(o_ref[...], wo_ref[...], preferred_element_type=jnp.float32)
          + bo_ref[...] + xres_ref[...])
    x2_ref[...] = x2
    h2 = _ln(x2, g_ref[...], b_ref[...])
    h2_ref[...] = h2
    logits = jnp.dot(h2, wr_ref[...], preferred_element_type=jnp.float32)
    col = lax.broadcasted_iota(jnp.int32, (_RT, 128), 1)
    lg = jnp.where(col < _E, logits, jnp.float32(-1e30))
    m = jnp.max(lg, axis=-1, keepdims=True)
    p = jnp.exp(lg - m)
    p = jnp.where(col < _E, p, 0.0)
    p = p / jnp.sum(p, axis=-1, keepdims=True)
    big = jnp.int32(1 << 30)
    m1 = jnp.max(p, axis=-1, keepdims=True)
    i1 = jnp.min(jnp.where(p == m1, col, big), axis=-1, keepdims=True)
    p2 = jnp.where(col == i1, jnp.float32(-1.0), p)
    m2 = jnp.max(p2, axis=-1, keepdims=True)
    i2 = jnp.min(jnp.where(p2 == m2, col, big), axis=-1, keepdims=True)
    den = m1 + m2
    g1 = m1 / den
    g2 = m2 / den
    rt = jnp.where(col == 0, i1.astype(jnp.float32),
         jnp.where(col == 1, i2.astype(jnp.float32),
         jnp.where(col == 2, g1,
         jnp.where(col == 3, g2, jnp.float32(0.0)))))
    rt_ref[...] = rt


def _proj_route(o2d, x2d, wo, bo, g, b, wr_pad):
    return pl.pallas_call(
        _proj_route_body,
        grid=(_T // _RT,),
        in_specs=[
            pl.BlockSpec((_RT, _D), lambda i: (i, 0)),
            pl.BlockSpec((_RT, _D), lambda i: (i, 0)),
            pl.BlockSpec((_D, _D), lambda i: (0, 0)),
            pl.BlockSpec((1, _D), lambda i: (0, 0)),
            pl.BlockSpec((1, _D), lambda i: (0, 0)),
            pl.BlockSpec((1, _D), lambda i: (0, 0)),
            pl.BlockSpec((_D, 128), lambda i: (0, 0)),
        ],
        out_specs=[
            pl.BlockSpec((_RT, _D), lambda i: (i, 0)),
            pl.BlockSpec((_RT, _D), lambda i: (i, 0)),
            pl.BlockSpec((_RT, 128), lambda i: (i, 0)),
        ],
        out_shape=[
            jax.ShapeDtypeStruct((_T, _D), jnp.float32),
            jax.ShapeDtypeStruct((_T, _D), jnp.float32),
            jax.ShapeDtypeStruct((_T, 128), jnp.float32),
        ],
    )(o2d, x2d, wo, bo.reshape(1, _D), g.reshape(1, _D), b.reshape(1, _D),
      wr_pad)


def _moe_body(te_ref, x_ref, w1_ref, b1_ref, w2_ref, b2_ref, gate_ref, y_ref):
    x = x_ref[...].astype(jnp.bfloat16)
    h = (jnp.dot(x, w1_ref[0], preferred_element_type=jnp.float32)
         + b1_ref[0])
    h = jax.nn.gelu(h).astype(jnp.bfloat16)
    y = jnp.dot(h, w2_ref[0], preferred_element_type=jnp.float32)
    y_ref[...] = (y + b2_ref[0]) * gate_ref[0]


def _moe_ffn(xs, w1, b1, w2, b2, gates, tile_expert):
    grid_spec = pltpu.PrefetchScalarGridSpec(
        num_scalar_prefetch=1,
        grid=(_NT,),
        in_specs=[
            pl.BlockSpec((_TILE, _D), lambda i, te: (i, 0)),
            pl.BlockSpec((1, _D, _F), lambda i, te: (te[i], 0, 0)),
            pl.BlockSpec((1, 1, _F), lambda i, te: (te[i], 0, 0)),
            pl.BlockSpec((1, _F, _D), lambda i, te: (te[i], 0, 0)),
            pl.BlockSpec((1, 1, _D), lambda i, te: (te[i], 0, 0)),
            pl.BlockSpec((1, _TILE, 1), lambda i, te: (i, 0, 0)),
        ],
        out_specs=pl.BlockSpec((_TILE, _D), lambda i, te: (i, 0)),
    )
    return pl.pallas_call(
        _moe_body,
        grid_spec=grid_spec,
        out_shape=jax.ShapeDtypeStruct((_P, _D), jnp.float32),
        compiler_params=pltpu.CompilerParams(
            vmem_limit_bytes=100 * 1024 * 1024),
    )(tile_expert, xs, w1.astype(jnp.bfloat16), b1.reshape(_E, 1, _F),
      w2.astype(jnp.bfloat16), b2.reshape(_E, 1, _D),
      gates.reshape(_NT, _TILE, 1))


def kernel(x, ln1_g, ln1_b, Wqkv, bqkv, Wo, bo, ln2_g, ln2_b, Wr, W1, b1,
           W2, b2):
    x2d = x.reshape(_T, _D)

    qkv = _ln_qkv(x2d, ln1_g, ln1_b, Wqkv, bqkv)
    o2d = _attention(qkv)

    wr_pad = jnp.zeros((_D, 128), jnp.float32).at[:, :_E].set(Wr)
    x2, h2, route = _proj_route(o2d, x2d, Wo, bo, ln2_g, ln2_b, wr_pad)

    i1 = route[:, 0].astype(jnp.int32)
    i2 = route[:, 1].astype(jnp.int32)
    g1 = route[:, 2]
    g2 = route[:, 3]

    e_pairs = jnp.stack([i1, i2], axis=1).reshape(-1)
    g_pairs = jnp.stack([g1, g2], axis=1).reshape(-1)
    onehot = (e_pairs[:, None] == jnp.arange(_E)[None, :]).astype(jnp.int32)
    carr = lax.associative_scan(jnp.add, onehot, axis=0)
    counts = carr[-1]
    rank = jnp.take_along_axis(carr, e_pairs[:, None], axis=1)[:, 0] - 1
    rc = ((counts + _TILE - 1) // _TILE) * _TILE
    ps = (jnp.cumsum(rc) - rc).astype(jnp.int32)
    dest = ps[e_pairs] + rank
    p1 = dest[0::2]
    p2 = dest[1::2]
    tok_pad = (jnp.zeros((_P,), jnp.int32).at[dest]
               .set(jnp.arange(_K * _T, dtype=jnp.int32) // _K))
    gate_pad = jnp.zeros((_P,), jnp.float32).at[dest].set(g_pairs)
    tile_starts = jnp.arange(_NT, dtype=jnp.int32) * _TILE
    tile_expert = (jnp.sum(ps[None, :] <= tile_starts[:, None], axis=1)
                   .astype(jnp.int32) - 1).clip(0, _E - 1)

    xs = jnp.take(h2, tok_pad, axis=0)
    ys = _moe_ffn(xs, W1, b1, W2, b2, gate_pad, tile_expert)
    out = x2 + jnp.take(ys, p1, axis=0) + jnp.take(ys, p2, axis=0)
    return out.reshape(_B, _S, _D)


# speedup vs baseline: 1.3532x; 1.0953x over previous
import functools

import jax
import jax.numpy as jnp
from jax import lax
from jax.experimental import pallas as pl
from jax.experimental.pallas import tpu as pltpu

_B, _S, _D = 2, 2048, 1024
_H, _Dh = 16, 64
_E, _K, _F = 8, 2, 4096
_T = _B * _S
_BH = _B * _H

_RT = 256
_BQ = 512
_BK = 512
_TILE = 256
_P = _K * _T + _E * _TILE
_NT = _P // _TILE
_FB = 2048
_NF = _F // _FB


def _ln(x, g, b):
    mu = jnp.mean(x, axis=-1, keepdims=True)
    xc = x - mu
    var = jnp.mean(xc * xc, axis=-1, keepdims=True)
    return xc * lax.rsqrt(var + 1e-5) * g + b


def _ln_qkv_body(x_ref, g_ref, b_ref, w_ref, bias_ref, o_ref):
    h = _ln(x_ref[...], g_ref[...], b_ref[...])
    o_ref[...] = (jnp.dot(h, w_ref[...], preferred_element_type=jnp.float32)
                  + bias_ref[...])


def _ln_qkv(x2d, g, b, w, bias):
    return pl.pallas_call(
        _ln_qkv_body,
        grid=(_T // _RT,),
        in_specs=[
            pl.BlockSpec((_RT, _D), lambda i: (i, 0)),
            pl.BlockSpec((1, _D), lambda i: (0, 0)),
            pl.BlockSpec((1, _D), lambda i: (0, 0)),
            pl.BlockSpec((_D, 3 * _D), lambda i: (0, 0)),
            pl.BlockSpec((1, 3 * _D), lambda i: (0, 0)),
        ],
        out_specs=pl.BlockSpec((_RT, 3 * _D), lambda i: (i, 0)),
        out_shape=jax.ShapeDtypeStruct((_T, 3 * _D), jnp.float32),
    )(x2d, g.reshape(1, _D), b.reshape(1, _D), w, bias.reshape(1, 3 * _D))


def _attn_body(q_ref, k_ref, v_ref, o_ref):
    qi = pl.program_id(2)
    row = qi * _BQ + lax.broadcasted_iota(jnp.int32, (_BQ, _BK), 0)
    col = lax.broadcasted_iota(jnp.int32, (_BQ, _BK), 1)
    for u in range(2):
        sl = pl.ds(u * _Dh, _Dh)
        q = q_ref[:, sl]

        def body(ki, carry):
            acc, m, l = carry
            ks = pl.ds(ki * _BK, _BK)
            k = k_ref[ks, sl]
            s = lax.dot_general(q, k, (((1,), (1,)), ((), ())),
                                preferred_element_type=jnp.float32) * 0.125
            s = jnp.where(ki * _BK + col <= row, s, jnp.float32(-1e9))
            mn = jnp.maximum(m, jnp.max(s, axis=-1, keepdims=True))
            p = jnp.exp(s - mn)
            scale = jnp.exp(m - mn)
            l = l * scale + jnp.sum(p, axis=-1, keepdims=True)
            acc = acc * scale + jnp.dot(p, v_ref[ks, sl],
                                        preferred_element_type=jnp.float32)
            return acc, mn, l

        acc, m, l = lax.fori_loop(
            0, qi * (_BQ // _BK) + 1, body,
            (jnp.zeros((_BQ, _Dh), jnp.float32),
             jnp.full((_BQ, 1), -1e30, jnp.float32),
             jnp.zeros((_BQ, 1), jnp.float32)))
        o_ref[:, sl] = acc / l


def _attention(qkv):
    nq = _S // _BQ
    nhp = _H // 2
    return pl.pallas_call(
        _attn_body,
        grid=(_B, nhp, nq),
        in_specs=[
            pl.BlockSpec((_BQ, 128), lambda b, hp, qi: (b * nq + qi, hp)),
            pl.BlockSpec((_S, 128), lambda b, hp, qi: (b, nhp + hp)),
            pl.BlockSpec((_S, 128), lambda b, hp, qi: (b, 2 * nhp + hp)),
        ],
        out_specs=pl.BlockSpec((_BQ, 128), lambda b, hp, qi: (b * nq + qi, hp)),
        out_shape=jax.ShapeDtypeStruct((_T, _D), jnp.float32),
    )(qkv, qkv, qkv)


def _proj_route_body(o_ref, xres_ref, wo_ref, bo_ref, g_ref, b_ref, wr_ref,
                     x2_ref, h2_ref, rt_ref):
    x2 = (jnp.dot(o_ref[...], wo_ref[...], preferred_element_type=jnp.float32)
          + bo_ref[...] + xres_ref[...])
    x2_ref[...] = x2
    h2 = _ln(x2, g_ref[...], b_ref[...])
    h2_ref[...] = h2
    logits = jnp.dot(h2, wr_ref[...], preferred_element_type=jnp.float32)
    col = lax.broadcasted_iota(jnp.int32, (_RT, 128), 1)
    lg = jnp.where(col < _E, logits, jnp.float32(-1e30))
    m = jnp.max(lg, axis=-1, keepdims=True)
    p = jnp.exp(lg - m)
    p = jnp.where(col < _E, p, 0.0)
    p = p / jnp.sum(p, axis=-1, keepdims=True)
    big = jnp.int32(1 << 30)
    m1 = jnp.max(p, axis=-1, keepdims=True)
    i1 = jnp.min(jnp.where(p == m1, col, big), axis=-1, keepdims=True)
    p2 = jnp.where(col == i1, jnp.float32(-1.0), p)
    m2 = jnp.max(p2, axis=-1, keepdims=True)
    i2 = jnp.min(jnp.where(p2 == m2, col, big), axis=-1, keepdims=True)
    den = m1 + m2
    g1 = m1 / den
    g2 = m2 / den
    rt = jnp.where(col == 0, i1.astype(jnp.float32),
         jnp.where(col == 1, i2.astype(jnp.float32),
         jnp.where(col == 2, g1,
         jnp.where(col == 3, g2, jnp.float32(0.0)))))
    rt_ref[...] = rt


def _proj_route(o2d, x2d, wo, bo, g, b, wr_pad):
    return pl.pallas_call(
        _proj_route_body,
        grid=(_T // _RT,),
        in_specs=[
            pl.BlockSpec((_RT, _D), lambda i: (i, 0)),
            pl.BlockSpec((_RT, _D), lambda i: (i, 0)),
            pl.BlockSpec((_D, _D), lambda i: (0, 0)),
            pl.BlockSpec((1, _D), lambda i: (0, 0)),
            pl.BlockSpec((1, _D), lambda i: (0, 0)),
            pl.BlockSpec((1, _D), lambda i: (0, 0)),
            pl.BlockSpec((_D, 128), lambda i: (0, 0)),
        ],
        out_specs=[
            pl.BlockSpec((_RT, _D), lambda i: (i, 0)),
            pl.BlockSpec((_RT, _D), lambda i: (i, 0)),
            pl.BlockSpec((_RT, 128), lambda i: (i, 0)),
        ],
        out_shape=[
            jax.ShapeDtypeStruct((_T, _D), jnp.float32),
            jax.ShapeDtypeStruct((_T, _D), jnp.float32),
            jax.ShapeDtypeStruct((_T, 128), jnp.float32),
        ],
    )(o2d, x2d, wo, bo.reshape(1, _D), g.reshape(1, _D), b.reshape(1, _D),
      wr_pad)


def _moe_body(te_ref, x_ref, w1_ref, b1_ref, w2_ref, b2_ref, gate_ref, y_ref):
    x = x_ref[...].astype(jnp.bfloat16)
    h = (jnp.dot(x, w1_ref[0], preferred_element_type=jnp.float32)
         + b1_ref[0])
    h = jax.nn.gelu(h).astype(jnp.bfloat16)
    y = jnp.dot(h, w2_ref[0], preferred_element_type=jnp.float32)
    y_ref[...] = (y + b2_ref[0]) * gate_ref[0]


def _moe_ffn(xs, w1, b1, w2, b2, gates, tile_expert):
    grid_spec = pltpu.PrefetchScalarGridSpec(
        num_scalar_prefetch=1,
        grid=(_NT,),
        in_specs=[
            pl.BlockSpec((_TILE, _D), lambda i, te: (i, 0)),
            pl.BlockSpec((1, _D, _F), lambda i, te: (te[i], 0, 0)),
            pl.BlockSpec((1, 1, _F), lambda i, te: (te[i], 0, 0)),
            pl.BlockSpec((1, _F, _D), lambda i, te: (te[i], 0, 0)),
            pl.BlockSpec((1, 1, _D), lambda i, te: (te[i], 0, 0)),
            pl.BlockSpec((1, _TILE, 1), lambda i, te: (i, 0, 0)),
        ],
        out_specs=pl.BlockSpec((_TILE, _D), lambda i, te: (i, 0)),
    )
    return pl.pallas_call(
        _moe_body,
        grid_spec=grid_spec,
        out_shape=jax.ShapeDtypeStruct((_P, _D), jnp.float32),
        compiler_params=pltpu.CompilerParams(
            vmem_limit_bytes=100 * 1024 * 1024),
    )(tile_expert, xs, w1.astype(jnp.bfloat16), b1.reshape(_E, 1, _F),
      w2.astype(jnp.bfloat16), b2.reshape(_E, 1, _D),
      gates.reshape(_NT, _TILE, 1))


def kernel(x, ln1_g, ln1_b, Wqkv, bqkv, Wo, bo, ln2_g, ln2_b, Wr, W1, b1,
           W2, b2):
    x2d = x.reshape(_T, _D)

    qkv = _ln_qkv(x2d, ln1_g, ln1_b, Wqkv, bqkv)
    o2d = _attention(qkv)

    wr_pad = jnp.zeros((_D, 128), jnp.float32).at[:, :_E].set(Wr)
    x2, h2, route = _proj_route(o2d, x2d, Wo, bo, ln2_g, ln2_b, wr_pad)

    i1 = route[:, 0].astype(jnp.int32)
    i2 = route[:, 1].astype(jnp.int32)
    g1 = route[:, 2]
    g2 = route[:, 3]

    e_pairs = jnp.stack([i1, i2], axis=1).reshape(-1)
    g_pairs = jnp.stack([g1, g2], axis=1).reshape(-1)
    onehot = (e_pairs[:, None] == jnp.arange(_E)[None, :]).astype(jnp.int32)
    carr = lax.associative_scan(jnp.add, onehot, axis=0)
    counts = carr[-1]
    rank = jnp.take_along_axis(carr, e_pairs[:, None], axis=1)[:, 0] - 1
    rc = ((counts + _TILE - 1) // _TILE) * _TILE
    ps = (jnp.cumsum(rc) - rc).astype(jnp.int32)
    dest = ps[e_pairs] + rank
    p1 = dest[0::2]
    p2 = dest[1::2]
    tok_pad = (jnp.zeros((_P,), jnp.int32).at[dest]
               .set(jnp.arange(_K * _T, dtype=jnp.int32) // _K))
    gate_pad = jnp.zeros((_P,), jnp.float32).at[dest].set(g_pairs)
    tile_starts = jnp.arange(_NT, dtype=jnp.int32) * _TILE
    tile_expert = (jnp.sum(ps[None, :] <= tile_starts[:, None], axis=1)
                   .astype(jnp.int32) - 1).clip(0, _E - 1)

    xs = jnp.take(h2, tok_pad, axis=0)
    ys = _moe_ffn(xs, W1, b1, W2, b2, gate_pad, tile_expert)
    out = x2 + jnp.take(ys, p1, axis=0) + jnp.take(ys, p2, axis=0)
    return out.reshape(_B, _S, _D)


# speedup vs baseline: 1.3609x; 1.0057x over previous
import functools

import jax
import jax.numpy as jnp
from jax import lax
from jax.experimental import pallas as pl
from jax.experimental.pallas import tpu as pltpu
from jax.experimental.pallas import tpu_sc as plsc

_B, _S, _D = 2, 2048, 1024
_H, _Dh = 16, 64
_E, _K, _F = 8, 2, 4096
_T = _B * _S
_BH = _B * _H

_RT = 256
_BQ = 512
_BK = 512
_TILE = 256
_P = _K * _T + _E * _TILE
_NT = _P // _TILE
_FB = 2048
_NF = _F // _FB


def _ln(x, g, b):
    mu = jnp.mean(x, axis=-1, keepdims=True)
    xc = x - mu
    var = jnp.mean(xc * xc, axis=-1, keepdims=True)
    return xc * lax.rsqrt(var + 1e-5) * g + b


def _ln_qkv_body(x_ref, g_ref, b_ref, w_ref, bias_ref, o_ref):
    h = _ln(x_ref[...], g_ref[...], b_ref[...])
    o_ref[...] = (jnp.dot(h, w_ref[...], preferred_element_type=jnp.float32)
                  + bias_ref[...])


def _ln_qkv(x2d, g, b, w, bias):
    return pl.pallas_call(
        _ln_qkv_body,
        grid=(_T // _RT,),
        in_specs=[
            pl.BlockSpec((_RT, _D), lambda i: (i, 0)),
            pl.BlockSpec((1, _D), lambda i: (0, 0)),
            pl.BlockSpec((1, _D), lambda i: (0, 0)),
            pl.BlockSpec((_D, 3 * _D), lambda i: (0, 0)),
            pl.BlockSpec((1, 3 * _D), lambda i: (0, 0)),
        ],
        out_specs=pl.BlockSpec((_RT, 3 * _D), lambda i: (i, 0)),
        out_shape=jax.ShapeDtypeStruct((_T, 3 * _D), jnp.float32),
    )(x2d, g.reshape(1, _D), b.reshape(1, _D), w, bias.reshape(1, 3 * _D))


def _attn_body(q_ref, k_ref, v_ref, o_ref):
    qi = pl.program_id(2)
    row = qi * _BQ + lax.broadcasted_iota(jnp.int32, (_BQ, _BK), 0)
    col = lax.broadcasted_iota(jnp.int32, (_BQ, _BK), 1)
    for u in range(2):
        sl = pl.ds(u * _Dh, _Dh)
        q = q_ref[:, sl]

        def body(ki, carry):
            acc, m, l = carry
            ks = pl.ds(ki * _BK, _BK)
            k = k_ref[ks, sl]
            s = lax.dot_general(q, k, (((1,), (1,)), ((), ())),
                                preferred_element_type=jnp.float32) * 0.125
            s = jnp.where(ki * _BK + col <= row, s, jnp.float32(-1e9))
            mn = jnp.maximum(m, jnp.max(s, axis=-1, keepdims=True))
            p = jnp.exp(s - mn)
            scale = jnp.exp(m - mn)
            l = l * scale + jnp.sum(p, axis=-1, keepdims=True)
            acc = acc * scale + jnp.dot(p, v_ref[ks, sl],
                                        preferred_element_type=jnp.float32)
            return acc, mn, l

        acc, m, l = lax.fori_loop(
            0, qi * (_BQ // _BK) + 1, body,
            (jnp.zeros((_BQ, _Dh), jnp.float32),
             jnp.full((_BQ, 1), -1e30, jnp.float32),
             jnp.zeros((_BQ, 1), jnp.float32)))
        o_ref[:, sl] = acc / l


def _attention(qkv):
    nq = _S // _BQ
    nhp = _H // 2
    return pl.pallas_call(
        _attn_body,
        grid=(_B, nhp, nq),
        in_specs=[
            pl.BlockSpec((_BQ, 128), lambda b, hp, qi: (b * nq + qi, hp)),
            pl.BlockSpec((_S, 128), lambda b, hp, qi: (b, nhp + hp)),
            pl.BlockSpec((_S, 128), lambda b, hp, qi: (b, 2 * nhp + hp)),
        ],
        out_specs=pl.BlockSpec((_BQ, 128), lambda b, hp, qi: (b * nq + qi, hp)),
        out_shape=jax.ShapeDtypeStruct((_T, _D), jnp.float32),
    )(qkv, qkv, qkv)


def _proj_route_body(o_ref, xres_ref, wo_ref, bo_ref, g_ref, b_ref, wr_ref,
                     x2_ref, h2_ref, rt_ref):
    x2 = (jnp.dot(o_ref[...], wo_ref[...], preferred_element_type=jnp.float32)
          + bo_ref[...] + xres_ref[...])
    x2_ref[...] = x2
    h2 = _ln(x2, g_ref[...], b_ref[...])
    h2_ref[...] = h2
    logits = jnp.dot(h2, wr_ref[...], preferred_element_type=jnp.float32)
    col = lax.broadcasted_iota(jnp.int32, (_RT, 128), 1)
    lg = jnp.where(col < _E, logits, jnp.float32(-1e30))
    m = jnp.max(lg, axis=-1, keepdims=True)
    p = jnp.exp(lg - m)
    p = jnp.where(col < _E, p, 0.0)
    p = p / jnp.sum(p, axis=-1, keepdims=True)
    big = jnp.int32(1 << 30)
    m1 = jnp.max(p, axis=-1, keepdims=True)
    i1 = jnp.min(jnp.where(p == m1, col, big), axis=-1, keepdims=True)
    p2 = jnp.where(col == i1, jnp.float32(-1.0), p)
    m2 = jnp.max(p2, axis=-1, keepdims=True)
    i2 = jnp.min(jnp.where(p2 == m2, col, big), axis=-1, keepdims=True)
    den = m1 + m2
    g1 = m1 / den
    g2 = m2 / den
    rt = jnp.where(col == 0, i1.astype(jnp.float32),
         jnp.where(col == 1, i2.astype(jnp.float32),
         jnp.where(col == 2, g1,
         jnp.where(col == 3, g2, jnp.float32(0.0)))))
    rt_ref[...] = rt


def _proj_route(o2d, x2d, wo, bo, g, b, wr_pad):
    return pl.pallas_call(
        _proj_route_body,
        grid=(_T // _RT,),
        in_specs=[
            pl.BlockSpec((_RT, _D), lambda i: (i, 0)),
            pl.BlockSpec((_RT, _D), lambda i: (i, 0)),
            pl.BlockSpec((_D, _D), lambda i: (0, 0)),
            pl.BlockSpec((1, _D), lambda i: (0, 0)),
            pl.BlockSpec((1, _D), lambda i: (0, 0)),
            pl.BlockSpec((1, _D), lambda i: (0, 0)),
            pl.BlockSpec((_D, 128), lambda i: (0, 0)),
        ],
        out_specs=[
            pl.BlockSpec((_RT, _D), lambda i: (i, 0)),
            pl.BlockSpec((_RT, _D), lambda i: (i, 0)),
            pl.BlockSpec((_RT, 128), lambda i: (i, 0)),
        ],
        out_shape=[
            jax.ShapeDtypeStruct((_T, _D), jnp.float32),
            jax.ShapeDtypeStruct((_T, _D), jnp.float32),
            jax.ShapeDtypeStruct((_T, 128), jnp.float32),
        ],
    )(o2d, x2d, wo, bo.reshape(1, _D), g.reshape(1, _D), b.reshape(1, _D),
      wr_pad)


def _moe_body(te_ref, x_ref, w1_ref, b1_ref, w2_ref, b2_ref, gate_ref, y_ref):
    x = x_ref[...].astype(jnp.bfloat16)
    h = (jnp.dot(x, w1_ref[0], preferred_element_type=jnp.float32)
         + b1_ref[0])
    h = jax.nn.gelu(h).astype(jnp.bfloat16)
    y = jnp.dot(h, w2_ref[0], preferred_element_type=jnp.float32)
    y_ref[...] = (y + b2_ref[0]) * gate_ref[0]


def _moe_ffn(xs, w1, b1, w2, b2, gates, tile_expert):
    grid_spec = pltpu.PrefetchScalarGridSpec(
        num_scalar_prefetch=1,
        grid=(_NT,),
        in_specs=[
            pl.BlockSpec((_TILE, _D), lambda i, te: (i, 0)),
            pl.BlockSpec((1, _D, _F), lambda i, te: (te[i], 0, 0)),
            pl.BlockSpec((1, 1, _F), lambda i, te: (te[i], 0, 0)),
            pl.BlockSpec((1, _F, _D), lambda i, te: (te[i], 0, 0)),
            pl.BlockSpec((1, 1, _D), lambda i, te: (te[i], 0, 0)),
            pl.BlockSpec((1, _TILE, 1), lambda i, te: (i, 0, 0)),
        ],
        out_specs=pl.BlockSpec((_TILE, _D), lambda i, te: (i, 0)),
    )
    return pl.pallas_call(
        _moe_body,
        grid_spec=grid_spec,
        out_shape=jax.ShapeDtypeStruct((_P, _D), jnp.float32),
        compiler_params=pltpu.CompilerParams(
            vmem_limit_bytes=100 * 1024 * 1024),
    )(tile_expert, xs, w1.astype(jnp.bfloat16), b1.reshape(_E, 1, _F),
      w2.astype(jnp.bfloat16), b2.reshape(_E, 1, _D),
      gates.reshape(_NT, _TILE, 1))


_NW = 32
_GW = _P // _NW
_GCH = 40
_CW = _T // _NW
_CCH = 32


def _sc_wid():
    return lax.axis_index("s") * 2 + lax.axis_index("c")


@functools.partial(
    pl.kernel,
    mesh=plsc.VectorSubcoreMesh(core_axis_name="c", subcore_axis_name="s"),
    out_type=jax.ShapeDtypeStruct((_P, _D), jnp.float32),
    scratch_types=[
        pltpu.VMEM((_GCH,), jnp.int32),
        pltpu.VMEM((_GCH, _D), jnp.float32),
        pltpu.SemaphoreType.DMA,
    ],
)
def _sc_dispatch(h2_hbm, tok_hbm, out_hbm, idx_v, rows_v, sem):
    base = _sc_wid() * _GW
    for c in range(_GW // _GCH):
        off = base + c * _GCH
        pltpu.sync_copy(tok_hbm.at[pl.ds(off, _GCH)], idx_v)
        pltpu.async_copy(h2_hbm.at[idx_v], rows_v, sem).wait()
        pltpu.sync_copy(rows_v, out_hbm.at[pl.ds(off, _GCH)])


@functools.partial(
    pl.kernel,
    mesh=plsc.VectorSubcoreMesh(core_axis_name="c", subcore_axis_name="s"),
    out_type=jax.ShapeDtypeStruct((_T, _D), jnp.float32),
    scratch_types=[
        pltpu.VMEM((_CCH,), jnp.int32),
        pltpu.VMEM((_CCH,), jnp.int32),
        pltpu.VMEM((_CCH, _D), jnp.float32),
        pltpu.VMEM((_CCH, _D), jnp.float32),
        pltpu.VMEM((_CCH, _D), jnp.float32),
        pltpu.SemaphoreType.DMA,
        pltpu.SemaphoreType.DMA,
    ],
)
def _sc_combine(ys_hbm, x2_hbm, p1_hbm, p2_hbm, out_hbm,
                i1_v, i2_v, y1_v, y2_v, xb_v, sem1, sem2):
    base = _sc_wid() * _CW
    for c in range(_CW // _CCH):
        off = base + c * _CCH
        pltpu.sync_copy(p1_hbm.at[pl.ds(off, _CCH)], i1_v)
        pltpu.sync_copy(p2_hbm.at[pl.ds(off, _CCH)], i2_v)
        cp1 = pltpu.async_copy(ys_hbm.at[i1_v], y1_v, sem1)
        cp2 = pltpu.async_copy(ys_hbm.at[i2_v], y2_v, sem2)
        pltpu.sync_copy(x2_hbm.at[pl.ds(off, _CCH)], xb_v)
        cp1.wait()
        cp2.wait()

        def outer(i, t):
            def inner(j, t2):
                slj = pl.ds(j * 16, 16)
                xb_v[i, slj] = xb_v[i, slj] + y1_v[i, slj] + y2_v[i, slj]
                return t2
            return lax.fori_loop(0, _D // 16, inner, t)

        lax.fori_loop(0, _CCH, outer, 0)
        pltpu.sync_copy(xb_v, out_hbm.at[pl.ds(off, _CCH)])


def kernel(x, ln1_g, ln1_b, Wqkv, bqkv, Wo, bo, ln2_g, ln2_b, Wr, W1, b1,
           W2, b2):
    x2d = x.reshape(_T, _D)

    qkv = _ln_qkv(x2d, ln1_g, ln1_b, Wqkv, bqkv)
    o2d = _attention(qkv)

    wr_pad = jnp.zeros((_D, 128), jnp.float32).at[:, :_E].set(Wr)
    x2, h2, route = _proj_route(o2d, x2d, Wo, bo, ln2_g, ln2_b, wr_pad)

    i1 = route[:, 0].astype(jnp.int32)
    i2 = route[:, 1].astype(jnp.int32)
    g1 = route[:, 2]
    g2 = route[:, 3]

    e_pairs = jnp.stack([i1, i2], axis=1).reshape(-1)
    g_pairs = jnp.stack([g1, g2], axis=1).reshape(-1)
    onehot = (e_pairs[:, None] == jnp.arange(_E)[None, :]).astype(jnp.int32)
    carr = lax.associative_scan(jnp.add, onehot, axis=0)
    counts = carr[-1]
    rank = jnp.take_along_axis(carr, e_pairs[:, None], axis=1)[:, 0] - 1
    rc = ((counts + _TILE - 1) // _TILE) * _TILE
    ps = (jnp.cumsum(rc) - rc).astype(jnp.int32)
    dest = ps[e_pairs] + rank
    p1 = dest[0::2]
    p2 = dest[1::2]
    tok_pad = (jnp.zeros((_P,), jnp.int32).at[dest]
               .set(jnp.arange(_K * _T, dtype=jnp.int32) // _K))
    gate_pad = jnp.zeros((_P,), jnp.float32).at[dest].set(g_pairs)
    tile_starts = jnp.arange(_NT, dtype=jnp.int32) * _TILE
    tile_expert = (jnp.sum(ps[None, :] <= tile_starts[:, None], axis=1)
                   .astype(jnp.int32) - 1).clip(0, _E - 1)

    xs = _sc_dispatch(h2, tok_pad)
    ys = _moe_ffn(xs, W1, b1, W2, b2, gate_pad, tile_expert)
    out = _sc_combine(ys, x2, p1, p2)
    return out.reshape(_B, _S, _D)


# speedup vs baseline: 1.3902x; 1.0215x over previous
import functools

import jax
import jax.numpy as jnp
from jax import lax
from jax.experimental import pallas as pl
from jax.experimental.pallas import tpu as pltpu
from jax.experimental.pallas import tpu_sc as plsc

_B, _S, _D = 2, 2048, 1024
_H, _Dh = 16, 64
_E, _K, _F = 8, 2, 4096
_T = _B * _S
_BH = _B * _H

_RT = 256
_BQ = 512
_BK = 512
_TILE = 256
_P = _K * _T + _E * _TILE
_NT = _P // _TILE
_FB = 2048
_NF = _F // _FB


def _ln(x, g, b):
    mu = jnp.mean(x, axis=-1, keepdims=True)
    xc = x - mu
    var = jnp.mean(xc * xc, axis=-1, keepdims=True)
    return xc * lax.rsqrt(var + 1e-5) * g + b


def _ln_qkv_body(x_ref, g_ref, b_ref, w_ref, bias_ref, o_ref):
    h = _ln(x_ref[...], g_ref[...], b_ref[...])
    o_ref[...] = (jnp.dot(h, w_ref[...], preferred_element_type=jnp.float32)
                  + bias_ref[...])


def _ln_qkv(x2d, g, b, w, bias):
    return pl.pallas_call(
        _ln_qkv_body,
        grid=(_T // _RT,),
        in_specs=[
            pl.BlockSpec((_RT, _D), lambda i: (i, 0)),
            pl.BlockSpec((1, _D), lambda i: (0, 0)),
            pl.BlockSpec((1, _D), lambda i: (0, 0)),
            pl.BlockSpec((_D, 3 * _D), lambda i: (0, 0)),
            pl.BlockSpec((1, 3 * _D), lambda i: (0, 0)),
        ],
        out_specs=pl.BlockSpec((_RT, 3 * _D), lambda i: (i, 0)),
        out_shape=jax.ShapeDtypeStruct((_T, 3 * _D), jnp.float32),
    )(x2d, g.reshape(1, _D), b.reshape(1, _D), w, bias.reshape(1, 3 * _D))


def _attn_body(q_ref, k_ref, v_ref, o_ref):
    qi = pl.program_id(2)
    row = qi * _BQ + lax.broadcasted_iota(jnp.int32, (_BQ, _BK), 0)
    col = lax.broadcasted_iota(jnp.int32, (_BQ, _BK), 1)
    for u in range(2):
        sl = pl.ds(u * _Dh, _Dh)
        q = q_ref[:, sl]

        def body(ki, carry):
            acc, m, l = carry
            ks = pl.ds(ki * _BK, _BK)
            k = k_ref[ks, sl]
            s = lax.dot_general(q, k, (((1,), (1,)), ((), ())),
                                preferred_element_type=jnp.float32) * 0.125
            s = jnp.where(ki * _BK + col <= row, s, jnp.float32(-1e9))
            mn = jnp.maximum(m, jnp.max(s, axis=-1, keepdims=True))
            p = jnp.exp(s - mn)
            scale = jnp.exp(m - mn)
            l = l * scale + jnp.sum(p, axis=-1, keepdims=True)
            acc = acc * scale + jnp.dot(p, v_ref[ks, sl],
                                        preferred_element_type=jnp.float32)
            return acc, mn, l

        acc, m, l = lax.fori_loop(
            0, qi * (_BQ // _BK) + 1, body,
            (jnp.zeros((_BQ, _Dh), jnp.float32),
             jnp.full((_BQ, 1), -1e30, jnp.float32),
             jnp.zeros((_BQ, 1), jnp.float32)))
        o_ref[:, sl] = acc / l


def _attention(qkv):
    nq = _S // _BQ
    nhp = _H // 2
    return pl.pallas_call(
        _attn_body,
        grid=(_B, nhp, nq),
        in_specs=[
            pl.BlockSpec((_BQ, 128), lambda b, hp, qi: (b * nq + qi, hp)),
            pl.BlockSpec((_S, 128), lambda b, hp, qi: (b, nhp + hp)),
            pl.BlockSpec((_S, 128), lambda b, hp, qi: (b, 2 * nhp + hp)),
        ],
        out_specs=pl.BlockSpec((_BQ, 128), lambda b, hp, qi: (b * nq + qi, hp)),
        out_shape=jax.ShapeDtypeStruct((_T, _D), jnp.float32),
    )(qkv, qkv, qkv)


def _proj_route_body(o_ref, xres_ref, wo_ref, bo_ref, g_ref, b_ref, wr_ref,
                     x2_ref, h2_ref, rt_ref):
    x2 = (jnp.dot(o_ref[...], wo_ref[...], preferred_element_type=jnp.float32)
          + bo_ref[...] + xres_ref[...])
    x2_ref[...] = x2
    h2 = _ln(x2, g_ref[...], b_ref[...])
    h2_ref[...] = h2
    logits = jnp.dot(h2, wr_ref[...], preferred_element_type=jnp.float32)
    col = lax.broadcasted_iota(jnp.int32, (_RT, 128), 1)
    lg = jnp.where(col < _E, logits, jnp.float32(-1e30))
    m = jnp.max(lg, axis=-1, keepdims=True)
    p = jnp.exp(lg - m)
    p = jnp.where(col < _E, p, 0.0)
    p = p / jnp.sum(p, axis=-1, keepdims=True)
    big = jnp.int32(1 << 30)
    m1 = jnp.max(p, axis=-1, keepdims=True)
    i1 = jnp.min(jnp.where(p == m1, col, big), axis=-1, keepdims=True)
    p2 = jnp.where(col == i1, jnp.float32(-1.0), p)
    m2 = jnp.max(p2, axis=-1, keepdims=True)
    i2 = jnp.min(jnp.where(p2 == m2, col, big), axis=-1, keepdims=True)
    den = m1 + m2
    g1 = m1 / den
    g2 = m2 / den
    rt = jnp.where(col == 0, i1.astype(jnp.float32),
         jnp.where(col == 1, i2.astype(jnp.float32),
         jnp.where(col == 2, g1,
         jnp.where(col == 3, g2, jnp.float32(0.0)))))
    rt_ref[...] = rt


def _proj_route(o2d, x2d, wo, bo, g, b, wr_pad):
    return pl.pallas_call(
        _proj_route_body,
        grid=(_T // _RT,),
        in_specs=[
            pl.BlockSpec((_RT, _D), lambda i: (i, 0)),
            pl.BlockSpec((_RT, _D), lambda i: (i, 0)),
            pl.BlockSpec((_D, _D), lambda i: (0, 0)),
            pl.BlockSpec((1, _D), lambda i: (0, 0)),
            pl.BlockSpec((1, _D), lambda i: (0, 0)),
            pl.BlockSpec((1, _D), lambda i: (0, 0)),
            pl.BlockSpec((_D, 128), lambda i: (0, 0)),
        ],
        out_specs=[
            pl.BlockSpec((_RT, _D), lambda i: (i, 0)),
            pl.BlockSpec((_RT, _D), lambda i: (i, 0)),
            pl.BlockSpec((_RT, 128), lambda i: (i, 0)),
        ],
        out_shape=[
            jax.ShapeDtypeStruct((_T, _D), jnp.float32),
            jax.ShapeDtypeStruct((_T, _D), jnp.float32),
            jax.ShapeDtypeStruct((_T, 128), jnp.float32),
        ],
    )(o2d, x2d, wo, bo.reshape(1, _D), g.reshape(1, _D), b.reshape(1, _D),
      wr_pad)


def _moe_body(te_ref, x_ref, w1_ref, b1_ref, w2_ref, b2_ref, gate_ref, y_ref):
    x = x_ref[...].astype(jnp.bfloat16)
    h = (jnp.dot(x, w1_ref[0], preferred_element_type=jnp.float32)
         + b1_ref[0])
    h = jax.nn.gelu(h).astype(jnp.bfloat16)
    y = jnp.dot(h, w2_ref[0], preferred_element_type=jnp.float32)
    y_ref[...] = (y + b2_ref[0]) * gate_ref[0]


def _moe_ffn(xs, w1, b1, w2, b2, gates, tile_expert):
    grid_spec = pltpu.PrefetchScalarGridSpec(
        num_scalar_prefetch=1,
        grid=(_NT,),
        in_specs=[
            pl.BlockSpec((_TILE, _D), lambda i, te: (i, 0)),
            pl.BlockSpec((1, _D, _F), lambda i, te: (te[i], 0, 0)),
            pl.BlockSpec((1, 1, _F), lambda i, te: (te[i], 0, 0)),
            pl.BlockSpec((1, _F, _D), lambda i, te: (te[i], 0, 0)),
            pl.BlockSpec((1, 1, _D), lambda i, te: (te[i], 0, 0)),
            pl.BlockSpec((1, _TILE, 1), lambda i, te: (i, 0, 0)),
        ],
        out_specs=pl.BlockSpec((_TILE, _D), lambda i, te: (i, 0)),
    )
    return pl.pallas_call(
        _moe_body,
        grid_spec=grid_spec,
        out_shape=jax.ShapeDtypeStruct((_P, _D), jnp.float32),
        compiler_params=pltpu.CompilerParams(
            vmem_limit_bytes=100 * 1024 * 1024),
    )(tile_expert, xs, w1.astype(jnp.bfloat16), b1.reshape(_E, 1, _F),
      w2.astype(jnp.bfloat16), b2.reshape(_E, 1, _D),
      gates.reshape(_NT, _TILE, 1))


_NW = 32
_GW = _P // _NW
_GCH = 40
_CW = _T // _NW
_CCH = 32


def _sc_wid():
    return lax.axis_index("s") * 2 + lax.axis_index("c")


@functools.partial(
    pl.kernel,
    mesh=plsc.VectorSubcoreMesh(core_axis_name="c", subcore_axis_name="s"),
    out_type=jax.ShapeDtypeStruct((_P, _D), jnp.float32),
    scratch_types=[
        pltpu.VMEM((_GW,), jnp.int32),
        pltpu.VMEM((_GCH, _D), jnp.float32),
        pltpu.VMEM((_GCH, _D), jnp.float32),
        pltpu.SemaphoreType.DMA,
        pltpu.SemaphoreType.DMA,
        pltpu.SemaphoreType.DMA,
        pltpu.SemaphoreType.DMA,
    ],
)
def _sc_dispatch(h2_hbm, tok_hbm, out_hbm, idx_v, rows0_v, rows1_v,
                 g0, g1, s0, s1):
    base = _sc_wid() * _GW
    pltpu.sync_copy(tok_hbm.at[pl.ds(base, _GW)], idx_v)
    bufs = (rows0_v, rows1_v)
    gsem = (g0, g1)
    ssem = (s0, s1)
    nch = _GW // _GCH
    gcp = [None, None]
    scp = [None, None]
    for c in range(2):
        gcp[c] = pltpu.async_copy(
            h2_hbm.at[idx_v.at[pl.ds(c * _GCH, _GCH)]], bufs[c], gsem[c])
    for c in range(nch):
        b = c % 2
        gcp[b].wait()
        scp[b] = pltpu.async_copy(
            bufs[b], out_hbm.at[pl.ds(base + c * _GCH, _GCH)], ssem[b])
        nxt = c + 2
        if nxt < nch:
            scp[b].wait()
            gcp[b] = pltpu.async_copy(
                h2_hbm.at[idx_v.at[pl.ds(nxt * _GCH, _GCH)]], bufs[b],
                gsem[b])
    scp[(nch - 2) % 2].wait()
    scp[(nch - 1) % 2].wait()


@functools.partial(
    pl.kernel,
    mesh=plsc.VectorSubcoreMesh(core_axis_name="c", subcore_axis_name="s"),
    out_type=jax.ShapeDtypeStruct((_T, _D), jnp.float32),
    scratch_types=[
        pltpu.VMEM((_CCH,), jnp.int32),
        pltpu.VMEM((_CCH,), jnp.int32),
        pltpu.VMEM((_CCH, _D), jnp.float32),
        pltpu.VMEM((_CCH, _D), jnp.float32),
        pltpu.VMEM((_CCH, _D), jnp.float32),
        pltpu.SemaphoreType.DMA,
        pltpu.SemaphoreType.DMA,
    ],
)
def _sc_combine(ys_hbm, x2_hbm, p1_hbm, p2_hbm, out_hbm,
                i1_v, i2_v, y1_v, y2_v, xb_v, sem1, sem2):
    base = _sc_wid() * _CW
    for c in range(_CW // _CCH):
        off = base + c * _CCH
        pltpu.sync_copy(p1_hbm.at[pl.ds(off, _CCH)], i1_v)
        pltpu.sync_copy(p2_hbm.at[pl.ds(off, _CCH)], i2_v)
        cp1 = pltpu.async_copy(ys_hbm.at[i1_v], y1_v, sem1)
        cp2 = pltpu.async_copy(ys_hbm.at[i2_v], y2_v, sem2)
        pltpu.sync_copy(x2_hbm.at[pl.ds(off, _CCH)], xb_v)
        cp1.wait()
        cp2.wait()

        def outer(i, t):
            for j in range(_D // 16):
                slj = pl.ds(j * 16, 16)
                xb_v[i, slj] = xb_v[i, slj] + y1_v[i, slj] + y2_v[i, slj]
            return t

        lax.fori_loop(0, _CCH, outer, 0)
        pltpu.sync_copy(xb_v, out_hbm.at[pl.ds(off, _CCH)])


def kernel(x, ln1_g, ln1_b, Wqkv, bqkv, Wo, bo, ln2_g, ln2_b, Wr, W1, b1,
           W2, b2):
    x2d = x.reshape(_T, _D)

    qkv = _ln_qkv(x2d, ln1_g, ln1_b, Wqkv, bqkv)
    o2d = _attention(qkv)

    wr_pad = jnp.zeros((_D, 128), jnp.float32).at[:, :_E].set(Wr)
    x2, h2, route = _proj_route(o2d, x2d, Wo, bo, ln2_g, ln2_b, wr_pad)

    i1 = route[:, 0].astype(jnp.int32)
    i2 = route[:, 1].astype(jnp.int32)
    g1 = route[:, 2]
    g2 = route[:, 3]

    e_pairs = jnp.stack([i1, i2], axis=1).reshape(-1)
    g_pairs = jnp.stack([g1, g2], axis=1).reshape(-1)
    onehot = (e_pairs[:, None] == jnp.arange(_E)[None, :]).astype(jnp.int32)
    carr = lax.associative_scan(jnp.add, onehot, axis=0)
    counts = carr[-1]
    rank = jnp.take_along_axis(carr, e_pairs[:, None], axis=1)[:, 0] - 1
    rc = ((counts + _TILE - 1) // _TILE) * _TILE
    ps = (jnp.cumsum(rc) - rc).astype(jnp.int32)
    dest = ps[e_pairs] + rank
    p1 = dest[0::2]
    p2 = dest[1::2]
    tok_pad = (jnp.zeros((_P,), jnp.int32).at[dest]
               .set(jnp.arange(_K * _T, dtype=jnp.int32) // _K))
    gate_pad = jnp.zeros((_P,), jnp.float32).at[dest].set(g_pairs)
    tile_starts = jnp.arange(_NT, dtype=jnp.int32) * _TILE
    tile_expert = (jnp.sum(ps[None, :] <= tile_starts[:, None], axis=1)
                   .astype(jnp.int32) - 1).clip(0, _E - 1)

    xs = _sc_dispatch(h2, tok_pad)
    ys = _moe_ffn(xs, W1, b1, W2, b2, gate_pad, tile_expert)
    out = _sc_combine(ys, x2, p1, p2)
    return out.reshape(_B, _S, _D)


# speedup vs baseline: 1.4553x; 1.0469x over previous
import functools

import jax
import jax.numpy as jnp
from jax import lax
from jax.experimental import pallas as pl
from jax.experimental.pallas import tpu as pltpu
from jax.experimental.pallas import tpu_sc as plsc

_B, _S, _D = 2, 2048, 1024
_H, _Dh = 16, 64
_E, _K, _F = 8, 2, 4096
_T = _B * _S
_BH = _B * _H

_RT = 256
_BQ = 512
_BK = 512
_TILE = 256
_P = _K * _T + _E * _TILE
_NT = _P // _TILE
_FB = 2048
_NF = _F // _FB


def _ln(x, g, b):
    mu = jnp.mean(x, axis=-1, keepdims=True)
    xc = x - mu
    var = jnp.mean(xc * xc, axis=-1, keepdims=True)
    return xc * lax.rsqrt(var + 1e-5) * g + b


def _ln_qkv_body(x_ref, g_ref, b_ref, w_ref, bias_ref, o_ref):
    h = _ln(x_ref[...], g_ref[...], b_ref[...])
    o_ref[...] = (jnp.dot(h, w_ref[...], preferred_element_type=jnp.float32)
                  + bias_ref[...])


def _ln_qkv(x2d, g, b, w, bias):
    return pl.pallas_call(
        _ln_qkv_body,
        grid=(_T // _RT,),
        in_specs=[
            pl.BlockSpec((_RT, _D), lambda i: (i, 0)),
            pl.BlockSpec((1, _D), lambda i: (0, 0)),
            pl.BlockSpec((1, _D), lambda i: (0, 0)),
            pl.BlockSpec((_D, 3 * _D), lambda i: (0, 0)),
            pl.BlockSpec((1, 3 * _D), lambda i: (0, 0)),
        ],
        out_specs=pl.BlockSpec((_RT, 3 * _D), lambda i: (i, 0)),
        out_shape=jax.ShapeDtypeStruct((_T, 3 * _D), jnp.float32),
    )(x2d, g.reshape(1, _D), b.reshape(1, _D), w, bias.reshape(1, 3 * _D))


def _attn_body(q_ref, k_ref, v_ref, o_ref):
    qi = pl.program_id(2)
    row = qi * _BQ + lax.broadcasted_iota(jnp.int32, (_BQ, _BK), 0)
    col = lax.broadcasted_iota(jnp.int32, (_BQ, _BK), 1)
    for u in range(2):
        sl = pl.ds(u * _Dh, _Dh)
        q = q_ref[:, sl]

        def body(ki, carry):
            acc, m, l = carry
            ks = pl.ds(ki * _BK, _BK)
            k = k_ref[ks, sl]
            s = lax.dot_general(q, k, (((1,), (1,)), ((), ())),
                                preferred_element_type=jnp.float32) * 0.125
            s = jnp.where(ki * _BK + col <= row, s, jnp.float32(-1e9))
            mn = jnp.maximum(m, jnp.max(s, axis=-1, keepdims=True))
            p = jnp.exp(s - mn)
            scale = jnp.exp(m - mn)
            l = l * scale + jnp.sum(p, axis=-1, keepdims=True)
            acc = acc * scale + jnp.dot(p, v_ref[ks, sl],
                                        preferred_element_type=jnp.float32)
            return acc, mn, l

        acc, m, l = lax.fori_loop(
            0, qi * (_BQ // _BK) + 1, body,
            (jnp.zeros((_BQ, _Dh), jnp.float32),
             jnp.full((_BQ, 1), -1e30, jnp.float32),
             jnp.zeros((_BQ, 1), jnp.float32)))
        o_ref[:, sl] = acc / l


def _attention(qkv):
    nq = _S // _BQ
    nhp = _H // 2
    return pl.pallas_call(
        _attn_body,
        grid=(_B, nhp, nq),
        in_specs=[
            pl.BlockSpec((_BQ, 128), lambda b, hp, qi: (b * nq + qi, hp)),
            pl.BlockSpec((_S, 128), lambda b, hp, qi: (b, nhp + hp)),
            pl.BlockSpec((_S, 128), lambda b, hp, qi: (b, 2 * nhp + hp)),
        ],
        out_specs=pl.BlockSpec((_BQ, 128), lambda b, hp, qi: (b * nq + qi, hp)),
        out_shape=jax.ShapeDtypeStruct((_T, _D), jnp.float32),
    )(qkv, qkv, qkv)


def _proj_route_body(o_ref, xres_ref, wo_ref, bo_ref, g_ref, b_ref, wr_ref,
                     x2_ref, h2_ref, rt_ref):
    x2 = (jnp.dot(o_ref[...], wo_ref[...], preferred_element_type=jnp.float32)
          + bo_ref[...] + xres_ref[...])
    x2_ref[...] = x2
    h2 = _ln(x2, g_ref[...], b_ref[...])
    h2_ref[...] = h2
    logits = jnp.dot(h2, wr_ref[...], preferred_element_type=jnp.float32)
    col = lax.broadcasted_iota(jnp.int32, (_RT, 128), 1)
    lg = jnp.where(col < _E, logits, jnp.float32(-1e30))
    m = jnp.max(lg, axis=-1, keepdims=True)
    p = jnp.exp(lg - m)
    p = jnp.where(col < _E, p, 0.0)
    p = p / jnp.sum(p, axis=-1, keepdims=True)
    big = jnp.int32(1 << 30)
    m1 = jnp.max(p, axis=-1, keepdims=True)
    i1 = jnp.min(jnp.where(p == m1, col, big), axis=-1, keepdims=True)
    p2 = jnp.where(col == i1, jnp.float32(-1.0), p)
    m2 = jnp.max(p2, axis=-1, keepdims=True)
    i2 = jnp.min(jnp.where(p2 == m2, col, big), axis=-1, keepdims=True)
    den = m1 + m2
    g1 = m1 / den
    g2 = m2 / den
    rt = jnp.where(col == 0, i1.astype(jnp.float32),
         jnp.where(col == 1, i2.astype(jnp.float32),
         jnp.where(col == 2, g1,
         jnp.where(col == 3, g2, jnp.float32(0.0)))))
    rt_ref[...] = rt


def _proj_route(o2d, x2d, wo, bo, g, b, wr_pad):
    return pl.pallas_call(
        _proj_route_body,
        grid=(_T // _RT,),
        in_specs=[
            pl.BlockSpec((_RT, _D), lambda i: (i, 0)),
            pl.BlockSpec((_RT, _D), lambda i: (i, 0)),
            pl.BlockSpec((_D, _D), lambda i: (0, 0)),
            pl.BlockSpec((1, _D), lambda i: (0, 0)),
            pl.BlockSpec((1, _D), lambda i: (0, 0)),
            pl.BlockSpec((1, _D), lambda i: (0, 0)),
            pl.BlockSpec((_D, 128), lambda i: (0, 0)),
        ],
        out_specs=[
            pl.BlockSpec((_RT, _D), lambda i: (i, 0)),
            pl.BlockSpec((_RT, _D), lambda i: (i, 0)),
            pl.BlockSpec((_RT, 128), lambda i: (i, 0)),
        ],
        out_shape=[
            jax.ShapeDtypeStruct((_T, _D), jnp.float32),
            jax.ShapeDtypeStruct((_T, _D), jnp.float32),
            jax.ShapeDtypeStruct((_T, 128), jnp.float32),
        ],
    )(o2d, x2d, wo, bo.reshape(1, _D), g.reshape(1, _D), b.reshape(1, _D),
      wr_pad)


def _moe_body(te_ref, x_ref, w1_ref, b1_ref, w2_ref, b2_ref, gate_ref, y_ref):
    x = x_ref[...].astype(jnp.bfloat16)
    h = (jnp.dot(x, w1_ref[0], preferred_element_type=jnp.float32)
         + b1_ref[0])
    h = jax.nn.gelu(h).astype(jnp.bfloat16)
    y = jnp.dot(h, w2_ref[0], preferred_element_type=jnp.float32)
    y_ref[...] = (y + b2_ref[0]) * gate_ref[0]


def _moe_ffn(xs, w1, b1, w2, b2, gates, tile_expert):
    grid_spec = pltpu.PrefetchScalarGridSpec(
        num_scalar_prefetch=1,
        grid=(_NT,),
        in_specs=[
            pl.BlockSpec((_TILE, _D), lambda i, te: (i, 0)),
            pl.BlockSpec((1, _D, _F), lambda i, te: (te[i], 0, 0)),
            pl.BlockSpec((1, 1, _F), lambda i, te: (te[i], 0, 0)),
            pl.BlockSpec((1, _F, _D), lambda i, te: (te[i], 0, 0)),
            pl.BlockSpec((1, 1, _D), lambda i, te: (te[i], 0, 0)),
            pl.BlockSpec((1, _TILE, 1), lambda i, te: (i, 0, 0)),
        ],
        out_specs=pl.BlockSpec((_TILE, _D), lambda i, te: (i, 0)),
    )
    return pl.pallas_call(
        _moe_body,
        grid_spec=grid_spec,
        out_shape=jax.ShapeDtypeStruct((_P, _D), jnp.float32),
        compiler_params=pltpu.CompilerParams(
            vmem_limit_bytes=100 * 1024 * 1024),
    )(tile_expert, xs, w1.astype(jnp.bfloat16), b1.reshape(_E, 1, _F),
      w2.astype(jnp.bfloat16), b2.reshape(_E, 1, _D),
      gates.reshape(_NT, _TILE, 1))


_NW = 32
_GW = _P // _NW
_GCH = 40
_CW = _T // _NW
_CCH = 32


def _sc_wid():
    return lax.axis_index("s") * 2 + lax.axis_index("c")


@functools.partial(
    pl.kernel,
    mesh=plsc.VectorSubcoreMesh(core_axis_name="c", subcore_axis_name="s"),
    out_type=jax.ShapeDtypeStruct((_P, _D), jnp.float32),
    scratch_types=[
        pltpu.VMEM((_GW,), jnp.int32),
        pltpu.VMEM((_GCH, _D), jnp.float32),
        pltpu.VMEM((_GCH, _D), jnp.float32),
        pltpu.SemaphoreType.DMA,
        pltpu.SemaphoreType.DMA,
        pltpu.SemaphoreType.DMA,
        pltpu.SemaphoreType.DMA,
    ],
)
def _sc_dispatch(h2_hbm, tok_hbm, out_hbm, idx_v, rows0_v, rows1_v,
                 g0, g1, s0, s1):
    base = _sc_wid() * _GW
    pltpu.sync_copy(tok_hbm.at[pl.ds(base, _GW)], idx_v)
    bufs = (rows0_v, rows1_v)
    gsem = (g0, g1)
    ssem = (s0, s1)
    nch = _GW // _GCH
    gcp = [None, None]
    scp = [None, None]
    for c in range(2):
        gcp[c] = pltpu.async_copy(
            h2_hbm.at[idx_v.at[pl.ds(c * _GCH, _GCH)]], bufs[c], gsem[c])
    for c in range(nch):
        b = c % 2
        gcp[b].wait()
        scp[b] = pltpu.async_copy(
            bufs[b], out_hbm.at[pl.ds(base + c * _GCH, _GCH)], ssem[b])
        nxt = c + 2
        if nxt < nch:
            scp[b].wait()
            gcp[b] = pltpu.async_copy(
                h2_hbm.at[idx_v.at[pl.ds(nxt * _GCH, _GCH)]], bufs[b],
                gsem[b])
    scp[(nch - 2) % 2].wait()
    scp[(nch - 1) % 2].wait()


@functools.partial(
    pl.kernel,
    mesh=plsc.VectorSubcoreMesh(core_axis_name="c", subcore_axis_name="s"),
    out_type=jax.ShapeDtypeStruct((_T, _D), jnp.float32),
    scratch_types=[
        pltpu.VMEM((_CCH,), jnp.int32),
        pltpu.VMEM((_CCH,), jnp.int32),
        pltpu.VMEM((_CCH, _D), jnp.float32),
        pltpu.VMEM((_CCH, _D), jnp.float32),
        pltpu.VMEM((_CCH, _D), jnp.float32),
        pltpu.SemaphoreType.DMA,
        pltpu.SemaphoreType.DMA,
    ],
)
def _sc_combine(ys_hbm, x2_hbm, p1_hbm, p2_hbm, out_hbm,
                i1_v, i2_v, y1_v, y2_v, xb_v, sem1, sem2):
    base = _sc_wid() * _CW
    for c in range(_CW // _CCH):
        off = base + c * _CCH
        pltpu.sync_copy(p1_hbm.at[pl.ds(off, _CCH)], i1_v)
        pltpu.sync_copy(p2_hbm.at[pl.ds(off, _CCH)], i2_v)
        cp1 = pltpu.async_copy(ys_hbm.at[i1_v], y1_v, sem1)
        cp2 = pltpu.async_copy(ys_hbm.at[i2_v], y2_v, sem2)
        pltpu.sync_copy(x2_hbm.at[pl.ds(off, _CCH)], xb_v)
        cp1.wait()
        cp2.wait()

        def outer(i, t):
            for j in range(_D // 16):
                slj = pl.ds(j * 16, 16)
                xb_v[i, slj] = xb_v[i, slj] + y1_v[i, slj] + y2_v[i, slj]
            return t

        lax.fori_loop(0, _CCH, outer, 0)
        pltpu.sync_copy(xb_v, out_hbm.at[pl.ds(off, _CCH)])


def kernel(x, ln1_g, ln1_b, Wqkv, bqkv, Wo, bo, ln2_g, ln2_b, Wr, W1, b1,
           W2, b2):
    x2d = x.reshape(_T, _D)

    qkv = _ln_qkv(x2d, ln1_g, ln1_b, Wqkv, bqkv)
    o2d = _attention(qkv)

    wr_pad = jnp.zeros((_D, 128), jnp.float32).at[:, :_E].set(Wr)
    x2, h2, route = _proj_route(o2d, x2d, Wo, bo, ln2_g, ln2_b, wr_pad)

    i1 = route[:, 0].astype(jnp.int32)
    i2 = route[:, 1].astype(jnp.int32)
    g1 = route[:, 2]
    g2 = route[:, 3]

    e_pairs = jnp.stack([i1, i2], axis=1).reshape(-1)
    g_pairs = jnp.stack([g1, g2], axis=1).reshape(-1)
    onehot = (e_pairs[:, None] == jnp.arange(_E)[None, :]).astype(jnp.int32)
    carr = lax.associative_scan(jnp.add, onehot, axis=0)
    counts = carr[-1]
    rank = jnp.take_along_axis(carr, e_pairs[:, None], axis=1)[:, 0] - 1
    rc = ((counts + _TILE - 1) // _TILE) * _TILE
    ps = (jnp.cumsum(rc) - rc).astype(jnp.int32)
    dest = ps[e_pairs] + rank
    p1 = dest[0::2]
    p2 = dest[1::2]
    tok_pad = (jnp.zeros((_P,), jnp.int32).at[dest]
               .set(jnp.arange(_K * _T, dtype=jnp.int32) // _K))
    gate_pad = jnp.zeros((_P,), jnp.float32).at[dest].set(g_pairs)
    tile_starts = jnp.arange(_NT, dtype=jnp.int32) * _TILE
    tile_expert = (jnp.sum(ps[None, :] <= tile_starts[:, None], axis=1)
                   .astype(jnp.int32) - 1).clip(0, _E - 1)

    xs = jnp.take(h2, tok_pad, axis=0)
    ys = _moe_ffn(xs, W1, b1, W2, b2, gate_pad, tile_expert)
    out = _sc_combine(ys, x2, p1, p2)
    return out.reshape(_B, _S, _D)
